# baseline (speedup 1.0000x reference)
import jax
import jax.numpy as jnp
import numpy as np
from jax.experimental import pallas as pl
from jax.experimental.pallas import tpu as pltpu

_IN_CH = 96
_OUT_CH = 192
_HO = 112
_WO = 112
_G = 16

_COEF_TABLE = np.array([
    [0, 0, 0, 0],
    [0, 0, 0, 1],
    [0, 1, 0, -1],
    [0, 1, 0, 0],
    [0, 0, 1, -1],
    [0, 0, 1, 0],
    [0, 1, 1, -2],
    [0, 1, 1, -1],
    [1, -1, -1, 1],
    [1, -1, -1, 2],
    [1, 0, -1, 0],
    [1, 0, -1, 1],
    [1, -1, 0, 0],
    [1, -1, 0, 1],
    [1, 0, 0, -1],
    [1, 0, 0, 0],
], dtype=np.float32)


def _weff_kernel(w_ref, coef_ref, out_ref):
    w = w_ref[...].reshape(_OUT_CH * 4, 16)
    w = jax.nn.softmax(w, axis=-1)
    out_ref[...] = jnp.dot(w, coef_ref[...],
                           preferred_element_type=jnp.float32)


_CB = 8


def _deinterleave_kernel(r_ref, s_ref, in_ref, out_ref):
    for cc in range(_CB):
        v = in_ref[0, cc]
        for dy in range(3):
            t = jnp.dot(r_ref[dy], v, preferred_element_type=jnp.float32)
            for dx in range(3):
                out_ref[0, cc, dy * 3 + dx] = jnp.dot(
                    t, s_ref[dx],
                    preferred_element_type=jnp.float32).astype(jnp.bfloat16)


def _combine_kernel(sel_ref, weff_ref, *refs):
    del sel_ref
    taps = refs[:-1]
    out_ref = refs[-1]
    o = pl.program_id(1)
    for c in range(4 * _G):
        a = taps[2 * c][0, 0].astype(jnp.float32)
        b = taps[2 * c + 1][0, 0].astype(jnp.float32)
        row = o * (4 * _G) + c
        out_ref[0, c] = (weff_ref[row, 0]
                         + weff_ref[row, 1] * a
                         + weff_ref[row, 2] * b
                         + weff_ref[row, 3] * (a * b))


def _tap_spec(k):
    def index_map(n, o, sel_ref, weff_ref):
        return (n, sel_ref[o, k], 0, 0)
    return pl.BlockSpec((1, 1, _HO, _WO), index_map)


@jax.jit
def kernel(x, weights, selection):
    n, c, h, w = x.shape

    ch = (selection >> 16) & 0xFFFF
    ry = (selection >> 8) & 0xFF
    rx = selection & 0xFF
    plane_idx = (ch * 9 + ry * 3 + rx).astype(jnp.int32)
    plane_idx = plane_idx.reshape(_OUT_CH // _G, 8 * _G)

    weff = pl.pallas_call(
        _weff_kernel,
        out_shape=jax.ShapeDtypeStruct((4 * _OUT_CH, 4), jnp.float32),
    )(weights, jnp.asarray(_COEF_TABLE))

    rsel = np.zeros((3, _HO, h), dtype=np.float32)
    ssel = np.zeros((3, w, _WO), dtype=np.float32)
    for d in range(3):
        for i in range(_HO):
            r = 2 * i + d - 1
            if 0 <= r < h:
                rsel[d, i, r] = 1.0
        for j in range(_WO):
            q = 2 * j + d - 1
            if 0 <= q < w:
                ssel[d, q, j] = 1.0

    planes = pl.pallas_call(
        _deinterleave_kernel,
        grid=(n, c // _CB),
        in_specs=[
            pl.BlockSpec((3, _HO, h), lambda n_, c_: (0, 0, 0)),
            pl.BlockSpec((3, w, _WO), lambda n_, c_: (0, 0, 0)),
            pl.BlockSpec((1, _CB, h, w), lambda n_, c_: (n_, c_, 0, 0)),
        ],
        out_specs=pl.BlockSpec((1, _CB, 9, _HO, _WO),
                               lambda n_, c_: (n_, c_, 0, 0, 0)),
        out_shape=jax.ShapeDtypeStruct((n, c, 9, _HO, _WO), jnp.bfloat16),
        compiler_params=pltpu.CompilerParams(
            dimension_semantics=("parallel", "parallel"),
        ),
    )(jnp.asarray(rsel), jnp.asarray(ssel), x)
    src = planes.reshape(n, c * 9, _HO, _WO)

    grid_spec = pltpu.PrefetchScalarGridSpec(
        num_scalar_prefetch=2,
        grid=(n, _OUT_CH // _G),
        in_specs=[_tap_spec(k) for k in range(8 * _G)],
        out_specs=pl.BlockSpec((1, 4 * _G, _HO, _WO),
                               lambda n_, o, sel, wf: (n_, o, 0, 0)),
    )
    out = pl.pallas_call(
        _combine_kernel,
        grid_spec=grid_spec,
        out_shape=jax.ShapeDtypeStruct((n, 4 * _OUT_CH, _HO, _WO),
                                       jnp.float32),
        compiler_params=pltpu.CompilerParams(
            dimension_semantics=("parallel", "parallel"),
        ),
    )(plane_idx, weff, *([src] * (8 * _G)))
    return out


# speedup vs baseline: 1.0211x; 1.0211x over previous
import jax
import jax.numpy as jnp
import numpy as np
from jax.experimental import pallas as pl
from jax.experimental.pallas import tpu as pltpu

_IN_CH = 96
_OUT_CH = 192
_HO = 112
_WO = 112
_G = 16

_COEF_TABLE = np.array([
    [0, 0, 0, 0],
    [0, 0, 0, 1],
    [0, 1, 0, -1],
    [0, 1, 0, 0],
    [0, 0, 1, -1],
    [0, 0, 1, 0],
    [0, 1, 1, -2],
    [0, 1, 1, -1],
    [1, -1, -1, 1],
    [1, -1, -1, 2],
    [1, 0, -1, 0],
    [1, 0, -1, 1],
    [1, -1, 0, 0],
    [1, -1, 0, 1],
    [1, 0, 0, -1],
    [1, 0, 0, 0],
], dtype=np.float32)


def _weff_kernel(w_ref, coef_ref, out_ref):
    w = w_ref[...].reshape(_OUT_CH * 4, 16)
    w = jax.nn.softmax(w, axis=-1)
    out_ref[...] = jnp.dot(w, coef_ref[...],
                           preferred_element_type=jnp.float32)


_CB = 8


def _deinterleave_kernel(r_ref, s_ref, in_ref, out_ref):
    for cc in range(_CB):
        v = in_ref[0, cc]
        for dy in range(3):
            t = jnp.dot(r_ref[dy], v, preferred_element_type=jnp.float32)
            for dx in range(3):
                out_ref[0, cc * 9 + dy * 3 + dx] = jnp.dot(
                    t, s_ref[dx], preferred_element_type=jnp.float32)


def _combine_kernel(sel_ref, weff_ref, *refs):
    del sel_ref
    taps = refs[:-1]
    out_ref = refs[-1]
    o = pl.program_id(1)
    for c in range(4 * _G):
        a = taps[2 * c][0, 0]
        b = taps[2 * c + 1][0, 0]
        row = o * (4 * _G) + c
        out_ref[0, c] = (weff_ref[row, 0]
                         + weff_ref[row, 1] * a
                         + weff_ref[row, 2] * b
                         + weff_ref[row, 3] * (a * b))


def _tap_spec(k):
    def index_map(n, o, sel_ref, weff_ref):
        return (n, sel_ref[o, k], 0, 0)
    return pl.BlockSpec((1, 1, _HO, _WO), index_map)


@jax.jit
def kernel(x, weights, selection):
    n, c, h, w = x.shape

    ch = (selection >> 16) & 0xFFFF
    ry = (selection >> 8) & 0xFF
    rx = selection & 0xFF
    plane_idx = (ch * 9 + ry * 3 + rx).astype(jnp.int32)
    plane_idx = plane_idx.reshape(_OUT_CH // _G, 8 * _G)

    weff = pl.pallas_call(
        _weff_kernel,
        out_shape=jax.ShapeDtypeStruct((4 * _OUT_CH, 4), jnp.float32),
    )(weights, jnp.asarray(_COEF_TABLE))

    rsel = np.zeros((3, _HO, h), dtype=np.float32)
    ssel = np.zeros((3, w, _WO), dtype=np.float32)
    for d in range(3):
        for i in range(_HO):
            r = 2 * i + d - 1
            if 0 <= r < h:
                rsel[d, i, r] = 1.0
        for j in range(_WO):
            q = 2 * j + d - 1
            if 0 <= q < w:
                ssel[d, q, j] = 1.0

    planes = pl.pallas_call(
        _deinterleave_kernel,
        grid=(n, c // _CB),
        in_specs=[
            pl.BlockSpec((3, _HO, h), lambda n_, c_: (0, 0, 0)),
            pl.BlockSpec((3, w, _WO), lambda n_, c_: (0, 0, 0)),
            pl.BlockSpec((1, _CB, h, w), lambda n_, c_: (n_, c_, 0, 0)),
        ],
        out_specs=pl.BlockSpec((1, _CB * 9, _HO, _WO),
                               lambda n_, c_: (n_, c_, 0, 0)),
        out_shape=jax.ShapeDtypeStruct((n, c * 9, _HO, _WO), jnp.float32),
        compiler_params=pltpu.CompilerParams(
            dimension_semantics=("parallel", "parallel"),
        ),
    )(jnp.asarray(rsel), jnp.asarray(ssel), x)
    src = planes

    grid_spec = pltpu.PrefetchScalarGridSpec(
        num_scalar_prefetch=2,
        grid=(n, _OUT_CH // _G),
        in_specs=[_tap_spec(k) for k in range(8 * _G)],
        out_specs=pl.BlockSpec((1, 4 * _G, _HO, _WO),
                               lambda n_, o, sel, wf: (n_, o, 0, 0)),
    )
    out = pl.pallas_call(
        _combine_kernel,
        grid_spec=grid_spec,
        out_shape=jax.ShapeDtypeStruct((n, 4 * _OUT_CH, _HO, _WO),
                                       jnp.float32),
        compiler_params=pltpu.CompilerParams(
            dimension_semantics=("parallel", "parallel"),
        ),
    )(plane_idx, weff, *([src] * (8 * _G)))
    return out


# speedup vs baseline: 1.0288x; 1.0075x over previous
import jax
import jax.numpy as jnp
import numpy as np
from jax.experimental import pallas as pl
from jax.experimental.pallas import tpu as pltpu

_IN_CH = 96
_OUT_CH = 192
_HO = 112
_WO = 112
_G = 4

_COEF_TABLE = np.array([
    [0, 0, 0, 0],
    [0, 0, 0, 1],
    [0, 1, 0, -1],
    [0, 1, 0, 0],
    [0, 0, 1, -1],
    [0, 0, 1, 0],
    [0, 1, 1, -2],
    [0, 1, 1, -1],
    [1, -1, -1, 1],
    [1, -1, -1, 2],
    [1, 0, -1, 0],
    [1, 0, -1, 1],
    [1, -1, 0, 0],
    [1, -1, 0, 1],
    [1, 0, 0, -1],
    [1, 0, 0, 0],
], dtype=np.float32)


def _weff_kernel(w_ref, coef_ref, out_ref):
    w = w_ref[...].reshape(_OUT_CH * 4, 16)
    w = jax.nn.softmax(w, axis=-1)
    out_ref[...] = jnp.dot(w, coef_ref[...],
                           preferred_element_type=jnp.float32)


_CB = 8


def _deinterleave_kernel(r_ref, s_ref, in_ref, out_ref):
    for cc in range(_CB):
        v = in_ref[0, cc]
        for dy in range(3):
            t = jnp.dot(r_ref[dy], v, preferred_element_type=jnp.float32)
            for dx in range(3):
                out_ref[0, cc * 9 + dy * 3 + dx] = jnp.dot(
                    t, s_ref[dx], preferred_element_type=jnp.float32)


def _combine_kernel(sel_ref, weff_ref, *refs):
    del sel_ref
    taps = refs[:-1]
    out_ref = refs[-1]
    o = pl.program_id(0)
    for c in range(4 * _G):
        a = taps[2 * c][:, 0]
        b = taps[2 * c + 1][:, 0]
        row = o * (4 * _G) + c
        out_ref[:, c] = (weff_ref[row, 0]
                         + weff_ref[row, 1] * a
                         + weff_ref[row, 2] * b
                         + weff_ref[row, 3] * (a * b))


def _tap_spec(k, n):
    def index_map(o, sel_ref, weff_ref):
        return (0, sel_ref[o, k], 0, 0)
    return pl.BlockSpec((n, 1, _HO, _WO), index_map)


@jax.jit
def kernel(x, weights, selection):
    n, c, h, w = x.shape

    ch = (selection >> 16) & 0xFFFF
    ry = (selection >> 8) & 0xFF
    rx = selection & 0xFF
    plane_idx = (ch * 9 + ry * 3 + rx).astype(jnp.int32)
    plane_idx = plane_idx.reshape(_OUT_CH // _G, 8 * _G)

    weff = pl.pallas_call(
        _weff_kernel,
        out_shape=jax.ShapeDtypeStruct((4 * _OUT_CH, 4), jnp.float32),
    )(weights, jnp.asarray(_COEF_TABLE))

    rsel = np.zeros((3, _HO, h), dtype=np.float32)
    ssel = np.zeros((3, w, _WO), dtype=np.float32)
    for d in range(3):
        for i in range(_HO):
            r = 2 * i + d - 1
            if 0 <= r < h:
                rsel[d, i, r] = 1.0
        for j in range(_WO):
            q = 2 * j + d - 1
            if 0 <= q < w:
                ssel[d, q, j] = 1.0

    planes = pl.pallas_call(
        _deinterleave_kernel,
        grid=(n, c // _CB),
        in_specs=[
            pl.BlockSpec((3, _HO, h), lambda n_, c_: (0, 0, 0)),
            pl.BlockSpec((3, w, _WO), lambda n_, c_: (0, 0, 0)),
            pl.BlockSpec((1, _CB, h, w), lambda n_, c_: (n_, c_, 0, 0)),
        ],
        out_specs=pl.BlockSpec((1, _CB * 9, _HO, _WO),
                               lambda n_, c_: (n_, c_, 0, 0)),
        out_shape=jax.ShapeDtypeStruct((n, c * 9, _HO, _WO), jnp.float32),
        compiler_params=pltpu.CompilerParams(
            dimension_semantics=("parallel", "parallel"),
        ),
    )(jnp.asarray(rsel), jnp.asarray(ssel), x)
    src = planes

    grid_spec = pltpu.PrefetchScalarGridSpec(
        num_scalar_prefetch=2,
        grid=(_OUT_CH // _G,),
        in_specs=[_tap_spec(k, n) for k in range(8 * _G)],
        out_specs=pl.BlockSpec((n, 4 * _G, _HO, _WO),
                               lambda o, sel, wf: (0, o, 0, 0)),
    )
    out = pl.pallas_call(
        _combine_kernel,
        grid_spec=grid_spec,
        out_shape=jax.ShapeDtypeStruct((n, 4 * _OUT_CH, _HO, _WO),
                                       jnp.float32),
        compiler_params=pltpu.CompilerParams(
            dimension_semantics=("arbitrary",),
        ),
    )(plane_idx, weff, *([src] * (8 * _G)))
    return out
